# baseline (speedup 1.0000x reference)
import jax
import jax.numpy as jnp
from jax.experimental import pallas as pl

D_MODEL_ = 768
D_SAE_ = 24576
K_ = 64
BATCH_ = 64
BN_ = 2048


def _encode_body(x_ref, w_ref, b_ref, o_ref):
    acc = jnp.dot(x_ref[...], w_ref[...], preferred_element_type=jnp.float32)
    o_ref[...] = jnp.maximum(acc + b_ref[...], 0.0)


def _encode(x, W_enc, b_enc):
    grid = (D_SAE_ // BN_,)
    return pl.pallas_call(
        _encode_body,
        grid=grid,
        in_specs=[
            pl.BlockSpec((BATCH_, D_MODEL_), lambda j: (0, 0)),
            pl.BlockSpec((D_MODEL_, BN_), lambda j: (0, j)),
            pl.BlockSpec((1, BN_), lambda j: (0, j)),
        ],
        out_specs=pl.BlockSpec((BATCH_, BN_), lambda j: (0, j)),
        out_shape=jax.ShapeDtypeStruct((BATCH_, D_SAE_), jnp.float32),
    )(x, W_enc, b_enc.reshape(1, D_SAE_))


def kernel(x, W_enc, b_enc, W_dec, b_dec):
    pre_relu = _encode(x, W_enc, b_enc)
    topk_vals, topk_idx = jax.lax.top_k(pre_relu, K_)
    rows = jnp.arange(BATCH_)[:, None]
    z = jnp.zeros_like(pre_relu).at[rows, topk_idx].set(topk_vals)
    recon = z @ W_dec + b_dec
    return (z, recon)


# speedup vs baseline: 2.6637x; 2.6637x over previous
import functools

import jax
import jax.numpy as jnp
from jax import lax
from jax.experimental import pallas as pl
from jax.experimental.pallas import tpu as pltpu, tpu_sc as plsc

D_MODEL = 768
D_SAE = 24576
K = 64
BATCH = 64

BN = 2048
RB = 64
L = 16


def _encode_body(x_ref, w_ref, b_ref, o_ref):
    acc = jnp.dot(x_ref[...], w_ref[...], preferred_element_type=jnp.float32)
    o_ref[...] = jnp.maximum(acc + b_ref[...], 0.0)


def _encode(x, W_enc, b_enc):
    return pl.pallas_call(
        _encode_body,
        grid=(D_SAE // BN,),
        in_specs=[
            pl.BlockSpec((BATCH, D_MODEL), lambda j: (0, 0)),
            pl.BlockSpec((D_MODEL, BN), lambda j: (0, j)),
            pl.BlockSpec((1, BN), lambda j: (0, j)),
        ],
        out_specs=pl.BlockSpec((BATCH, BN), lambda j: (0, j)),
        out_shape=jax.ShapeDtypeStruct((BATCH, D_SAE), jnp.float32),
    )(x, W_enc, b_enc.reshape(1, D_SAE))


_NCH = D_SAE // 128
_NACC = 8


def _count_ge(b, cand):
    accs = [jnp.zeros((RB, 128), jnp.float32) for _ in range(_NACC)]
    for c in range(_NCH):
        ind = jnp.where(b[:, c * 128:(c + 1) * 128] >= cand, 1.0, 0.0)
        accs[c % _NACC] = accs[c % _NACC] + ind
    tot = accs[0]
    for acc in accs[1:]:
        tot = tot + acc
    return jnp.sum(tot, axis=1, keepdims=True)


def _select_body(p_ref, t_ref, r_ref):
    b = lax.bitcast_convert_type(p_ref[...], jnp.int32)

    def val_iter(i, t):
        cand = t | (1 << (30 - i))
        return jnp.where(_count_ge(b, cand) >= K, cand, t)

    t = lax.fori_loop(0, 31, val_iter, jnp.zeros((RB, 1), jnp.int32))
    a = _count_ge(b, t + 1)
    t_ref[...] = jnp.broadcast_to(
        lax.bitcast_convert_type(t, jnp.float32), (RB, L))
    r_ref[...] = jnp.broadcast_to(K - a.astype(jnp.int32), (RB, L))


def _select(pre):
    return pl.pallas_call(
        _select_body,
        grid=(BATCH // RB,),
        in_specs=[pl.BlockSpec((RB, D_SAE), lambda j: (j, 0))],
        out_specs=[
            pl.BlockSpec((RB, L), lambda j: (j, 0)),
            pl.BlockSpec((RB, L), lambda j: (j, 0)),
        ],
        out_shape=[
            jax.ShapeDtypeStruct((BATCH, L), jnp.float32),
            jax.ShapeDtypeStruct((BATCH, L), jnp.int32),
        ],
    )(pre)


def _make_k3():
    info = plsc.get_sparse_core_info()
    nc, ns = info.num_cores, info.num_subcores
    nw = nc * ns
    rows_per_w = BATCH // nw
    mesh = plsc.VectorSubcoreMesh(core_axis_name="c", subcore_axis_name="s")

    @functools.partial(
        pl.kernel,
        mesh=mesh,
        compiler_params=pltpu.CompilerParams(needs_layout_passes=False),
        out_type=[
            jax.ShapeDtypeStruct((BATCH, D_SAE), jnp.float32),
            jax.ShapeDtypeStruct((BATCH, D_MODEL), jnp.float32),
        ],
        scratch_types=[
            pltpu.VMEM((D_SAE,), jnp.float32),
            pltpu.VMEM((D_SAE,), jnp.float32),
            pltpu.VMEM((K,), jnp.int32),
            pltpu.VMEM((K,), jnp.float32),
            pltpu.VMEM((K, D_MODEL), jnp.float32),
            pltpu.VMEM((D_MODEL,), jnp.float32),
            pltpu.VMEM((D_MODEL,), jnp.float32),
            pltpu.VMEM((L,), jnp.float32),
            pltpu.VMEM((L,), jnp.int32),
            pltpu.SemaphoreType.DMA,
            pltpu.SemaphoreType.DMA,
            pltpu.SemaphoreType.DMA,
            pltpu.SemaphoreType.DMA,
            pltpu.SemaphoreType.DMA,
        ],
    )
    def k3(pre_hbm, tb_hbm, rr_hbm, wdec_hbm, bdec_hbm, z_hbm, recon_hbm,
           row0_v, row1_v, idxs_v, vals_v, rows_v, bdec_v, acc_v, tb_v, rr_v,
           sem_in0, sem_in1, sem_z0, sem_z1, sem_g):
        wid = lax.axis_index("s") * nc + lax.axis_index("c")
        row0 = wid * rows_per_w
        in0 = pltpu.async_copy(pre_hbm.at[row0], row0_v, sem_in0)
        in1 = pltpu.async_copy(pre_hbm.at[row0 + 1], row1_v, sem_in1)
        pltpu.sync_copy(bdec_hbm, bdec_v)

        zcopies = []
        for rr, (row_v, cin, sem_z) in enumerate(
                ((row0_v, in0, sem_z0), (row1_v, in1, sem_z1))):
            row = row0 + rr
            pltpu.sync_copy(tb_hbm.at[row], tb_v)
            pltpu.sync_copy(rr_hbm.at[row], rr_v)
            tsplat = tb_v[...]
            rsplat = rr_v[...]
            cin.wait()

            giota = lax.iota(jnp.int32, L)
            UNROLL = 8

            def body(ib, carry):
                cnt, erun = carry
                for j in range(UNROLL):
                    off = ib * (UNROLL * L) + j * L
                    v = row_v[pl.ds(off, L)]
                    gt = v > tsplat
                    eq = v == tsplat
                    rank = erun + plsc.cumsum(eq.astype(jnp.int32))
                    keep = gt | (eq & (rank <= rsplat))
                    row_v[pl.ds(off, L)] = jnp.where(keep, v, 0.0)
                    pos = jnp.maximum(
                        cnt + plsc.cumsum(keep.astype(jnp.int32)) - 1, 0)
                    plsc.store_scatter(idxs_v, [pos], giota + off, mask=keep)
                    plsc.store_scatter(vals_v, [pos], v, mask=keep)
                    cnt = cnt + plsc.all_reduce_population_count(keep)
                    erun = erun + plsc.all_reduce_population_count(eq)
                return cnt, erun

            lax.fori_loop(0, D_SAE // (UNROLL * L), body,
                          (jnp.zeros((L,), jnp.int32),
                           jnp.zeros((L,), jnp.int32)))

            zcopies.append(pltpu.async_copy(row_v, z_hbm.at[row], sem_z))
            pltpu.async_copy(wdec_hbm.at[idxs_v], rows_v, sem_g).wait()

            for g in range(3):
                def kbody(kb, accs, g=g):
                    vchunk = vals_v[pl.ds(kb * L, L)]
                    for kk in range(L):
                        vs = lax.gather(
                            vchunk,
                            jnp.full((L, 1), kk, jnp.int32),
                            lax.GatherDimensionNumbers(
                                offset_dims=(), collapsed_slice_dims=(0,),
                                start_index_map=(0,)),
                            slice_sizes=(1,),
                            mode=lax.GatherScatterMode.PROMISE_IN_BOUNDS)
                        k = kb * L + kk
                        accs = tuple(
                            accs[j] + vs * rows_v[k, pl.ds((g * 16 + j) * L, L)]
                            for j in range(16))
                    return accs

                init = tuple(bdec_v[pl.ds((g * 16 + j) * L, L)]
                             for j in range(16))
                accs = lax.fori_loop(0, K // L, kbody, init)
                for j in range(16):
                    acc_v[pl.ds((g * 16 + j) * L, L)] = accs[j]
            pltpu.sync_copy(acc_v, recon_hbm.at[row])
        for zc in zcopies:
            zc.wait()

    return k3


_k3 = _make_k3()


def kernel(x, W_enc, b_enc, W_dec, b_dec):
    pre = _encode(x, W_enc, b_enc)
    tb, rr = _select(pre)
    z, recon = _k3(pre, tb, rr, W_dec, b_dec)
    return (z, recon)
